# baseline (device time: 66436 ns/iter reference)
import functools

import jax
import jax.numpy as jnp
from jax import lax
from jax.experimental import pallas as pl
from jax.experimental.pallas import tpu as pltpu

N_DEV = 4
N_TOK = 2048
D_MODEL = 512
D_FF = 1024
HALF = D_FF // 2
N_EXP = 32
E_LOCAL = N_EXP // N_DEV
CHUNK = N_TOK // N_DEV
N_RS = N_DEV - 1
N_SUB = 2
SUB = CHUNK // N_SUB
N_SEM = N_RS + N_RS * N_SUB


def kernel(x, router_W, route_idx, expert_W):
    expert_W = expert_W.astype(jnp.bfloat16)

    def body(x_ref, rw_ref, idx_ref, ew_ref, out_ref,
             recv_buf, send_sems, recv_sems):
        p = lax.axis_index("i")
        left = lax.rem(p + N_DEV - 1, N_DEV)
        right = lax.rem(p + 1, N_DEV)

        def rowds(c):
            return pl.ds(lax.rem(c, N_DEV) * CHUNK, CHUNK)

        w_cat = ew_ref[...].reshape(E_LOCAL * D_MODEL, D_FF)

        def compute_chunk(c):
            rows = rowds(c)
            xc = x_ref[rows, :]
            scores = jnp.dot(xc, rw_ref[...],
                             preferred_element_type=jnp.float32)
            m = jnp.max(scores, axis=-1, keepdims=True)
            e = jnp.exp(scores - m)
            probs = e / jnp.sum(e, axis=-1, keepdims=True)

            idxc = idx_ref[rows, :]
            idx0 = idxc[:, 0:1]
            idx1 = idxc[:, 1:2]
            iota_e = lax.broadcasted_iota(jnp.int32, (CHUNK, N_EXP), 1)
            g0 = jnp.sum(jnp.where(iota_e == idx0, probs, 0.0),
                         axis=-1, keepdims=True)
            g1 = jnp.sum(jnp.where(iota_e == idx1, probs, 0.0),
                         axis=-1, keepdims=True)
            gs = g0 + g1
            local_ids = p * E_LOCAL + lax.broadcasted_iota(
                jnp.int32, (1, E_LOCAL), 1)
            w = (jnp.where(idx0 == local_ids, g0, 0.0)
                 + jnp.where(idx1 == local_ids, g1, 0.0)) / gs

            xb = xc.astype(jnp.bfloat16)
            wb = w.astype(jnp.bfloat16)
            x_cat = jnp.concatenate(
                [xb * wb[:, j:j + 1] for j in range(E_LOCAL)], axis=1)
            acc = jnp.dot(x_cat, w_cat, preferred_element_type=jnp.float32)
            out_ref[rows, :] = acc.astype(jnp.bfloat16)

        def ring_cols(ring):
            return pl.ds(0 if ring == 0 else HALF, HALF)

        def ring_tgt(ring):
            return right if ring == 0 else left

        def rs_rdma(ring, hop, chunk):
            return pltpu.make_async_remote_copy(
                src_ref=out_ref.at[rowds(chunk), ring_cols(ring)],
                dst_ref=recv_buf.at[ring, hop],
                send_sem=send_sems.at[ring, hop],
                recv_sem=recv_sems.at[ring, hop],
                device_id=(ring_tgt(ring),),
                device_id_type=pl.DeviceIdType.MESH,
            )

        def ag_rdma(ring, hop, sub, chunk):
            row0 = lax.rem(chunk, N_DEV) * CHUNK + sub * SUB
            sl = (pl.ds(row0, SUB), ring_cols(ring))
            idx = N_RS + hop * N_SUB + sub
            return pltpu.make_async_remote_copy(
                src_ref=out_ref.at[sl],
                dst_ref=out_ref.at[sl],
                send_sem=send_sems.at[ring, idx],
                recv_sem=recv_sems.at[ring, idx],
                device_id=(ring_tgt(ring),),
                device_id_type=pl.DeviceIdType.MESH,
            )

        def accumulate(ring, hop, chunk):
            sl = (rowds(chunk), ring_cols(ring))
            out_ref[sl] = out_ref[sl] + recv_buf[ring, hop]

        compute_chunk(p)

        bsem = pltpu.get_barrier_semaphore()
        for nbr in (left, right):
            pl.semaphore_signal(bsem, inc=1, device_id=(nbr,),
                                device_id_type=pl.DeviceIdType.MESH)
        pl.semaphore_wait(bsem, 2)

        ag_sends = []

        def ag_start(ring, hop, sub, chunk):
            r = ag_rdma(ring, hop, sub, chunk)
            r.start()
            ag_sends.append(r)

        for s in range(N_RS):
            ra = rs_rdma(0, s, p + N_DEV - s)
            rb = rs_rdma(1, s, p + s)
            ra.start()
            rb.start()
            if s == 0:
                compute_chunk(p + 1)
                compute_chunk(p + N_DEV - 1)
            elif s == 1:
                compute_chunk(p + 2)
            ra.wait()
            accumulate(0, s, p + N_DEV - s - 1)
            if s == N_RS - 1:
                for u in range(N_SUB):
                    ag_start(0, 0, u, p + N_DEV + 1)
            rb.wait()
            accumulate(1, s, p + s + 1)

        for u in range(N_SUB):
            ag_start(1, 0, u, p + N_DEV - 1)

        for t in range(1, N_RS):
            for u in range(N_SUB):
                ag_rdma(0, t - 1, u, p + N_DEV - (t - 1)).wait_recv()
                ag_start(0, t, u, p + N_DEV + 1 - t)
                ag_rdma(1, t - 1, u, p + (t - 1)).wait_recv()
                ag_start(1, t, u, p + N_DEV - 1 + t)
        for u in range(N_SUB):
            ag_rdma(0, N_RS - 1, u, p + N_DEV - (N_RS - 1)).wait_recv()
            ag_rdma(1, N_RS - 1, u, p + (N_RS - 1)).wait_recv()
        for r in ag_sends:
            r.wait_send()

        @functools.partial(pl.run_scoped, sem2=pltpu.SemaphoreType.REGULAR)
        def _(sem2):
            for nbr in (left, right):
                pl.semaphore_signal(sem2, inc=1, device_id=(nbr,),
                                    device_id_type=pl.DeviceIdType.MESH)
            pl.semaphore_wait(sem2, 2)

    return pl.pallas_call(
        body,
        out_shape=jax.ShapeDtypeStruct((N_TOK, D_FF), jnp.bfloat16),
        in_specs=[pl.BlockSpec(memory_space=pltpu.VMEM)] * 4,
        out_specs=pl.BlockSpec(memory_space=pltpu.VMEM),
        scratch_shapes=[
            pltpu.VMEM((2, N_RS, CHUNK, HALF), jnp.bfloat16),
            pltpu.SemaphoreType.DMA((2, N_SEM)),
            pltpu.SemaphoreType.DMA((2, N_SEM)),
        ],
        compiler_params=pltpu.CompilerParams(collective_id=0),
    )(x, router_W, route_idx, expert_W)


# device time: 64610 ns/iter; 1.0283x vs baseline; 1.0283x over previous
import functools

import jax
import jax.numpy as jnp
from jax import lax
from jax.experimental import pallas as pl
from jax.experimental.pallas import tpu as pltpu

N_DEV = 4
N_TOK = 2048
D_MODEL = 512
D_FF = 1024
HALF = D_FF // 2
N_EXP = 32
E_LOCAL = N_EXP // N_DEV
CHUNK = N_TOK // N_DEV
N_RS = N_DEV - 1
N_SUB = 2
SUB = CHUNK // N_SUB
N_SEM = N_RS + N_RS * N_SUB


def kernel(x, router_W, route_idx, expert_W):
    expert_W = expert_W.astype(jnp.bfloat16)

    def body(x_ref, rw_ref, idx_ref, ew_ref, out_ref,
             xcat_ref, recv_buf, send_sems, recv_sems):
        p = lax.axis_index("i")
        left = lax.rem(p + N_DEV - 1, N_DEV)
        right = lax.rem(p + 1, N_DEV)

        def rowds(c):
            return pl.ds(lax.rem(c, N_DEV) * CHUNK, CHUNK)

        w_cat = ew_ref[...].reshape(E_LOCAL * D_MODEL, D_FF)

        def prep_chunk(c):
            rows = rowds(c)
            xc = x_ref[rows, :]
            scores = jnp.dot(xc, rw_ref[...],
                             preferred_element_type=jnp.float32)
            m = jnp.max(scores, axis=-1, keepdims=True)
            e = jnp.exp(scores - m)
            probs = e / jnp.sum(e, axis=-1, keepdims=True)

            idxc = idx_ref[rows, :]
            idx0 = idxc[:, 0:1]
            idx1 = idxc[:, 1:2]
            iota_e = lax.broadcasted_iota(jnp.int32, (CHUNK, N_EXP), 1)
            g0 = jnp.sum(jnp.where(iota_e == idx0, probs, 0.0),
                         axis=-1, keepdims=True)
            g1 = jnp.sum(jnp.where(iota_e == idx1, probs, 0.0),
                         axis=-1, keepdims=True)
            gs = g0 + g1
            local_ids = p * E_LOCAL + lax.broadcasted_iota(
                jnp.int32, (1, E_LOCAL), 1)
            w = (jnp.where(idx0 == local_ids, g0, 0.0)
                 + jnp.where(idx1 == local_ids, g1, 0.0)) / gs

            xb = xc.astype(jnp.bfloat16)
            wb = w.astype(jnp.bfloat16)
            xcat_ref[rows, :] = jnp.concatenate(
                [xb * wb[:, j:j + 1] for j in range(E_LOCAL)], axis=1)

        def gemm_half(c, ring):
            rows = rowds(c)
            wc = w_cat[:, ring * HALF:(ring + 1) * HALF]
            acc = jnp.dot(xcat_ref[rows, :], wc,
                          preferred_element_type=jnp.float32)
            out_ref[rows, pl.ds(ring * HALF, HALF)] = acc.astype(jnp.bfloat16)

        def ring_cols(ring):
            return pl.ds(0 if ring == 0 else HALF, HALF)

        def ring_tgt(ring):
            return right if ring == 0 else left

        def rs_rdma(ring, hop, chunk):
            return pltpu.make_async_remote_copy(
                src_ref=out_ref.at[rowds(chunk), ring_cols(ring)],
                dst_ref=recv_buf.at[ring, hop],
                send_sem=send_sems.at[ring, hop],
                recv_sem=recv_sems.at[ring, hop],
                device_id=(ring_tgt(ring),),
                device_id_type=pl.DeviceIdType.MESH,
            )

        def ag_rdma(ring, hop, sub, chunk):
            row0 = lax.rem(chunk, N_DEV) * CHUNK + sub * SUB
            sl = (pl.ds(row0, SUB), ring_cols(ring))
            idx = N_RS + hop * N_SUB + sub
            return pltpu.make_async_remote_copy(
                src_ref=out_ref.at[sl],
                dst_ref=out_ref.at[sl],
                send_sem=send_sems.at[ring, idx],
                recv_sem=recv_sems.at[ring, idx],
                device_id=(ring_tgt(ring),),
                device_id_type=pl.DeviceIdType.MESH,
            )

        def accumulate(ring, hop, chunk):
            sl = (rowds(chunk), ring_cols(ring))
            out_ref[sl] = out_ref[sl] + recv_buf[ring, hop]

        ag_sends = []

        def ag_start(ring, hop, sub, chunk):
            r = ag_rdma(ring, hop, sub, chunk)
            r.start()
            ag_sends.append(r)

        prep_chunk(p)
        gemm_half(p, 0)

        bsem = pltpu.get_barrier_semaphore()
        for nbr in (left, right):
            pl.semaphore_signal(bsem, inc=1, device_id=(nbr,),
                                device_id_type=pl.DeviceIdType.MESH)
        pl.semaphore_wait(bsem, 2)

        ra = rs_rdma(0, 0, p)
        ra.start()
        gemm_half(p, 1)
        rb = rs_rdma(1, 0, p)
        rb.start()

        prep_chunk(p + N_DEV - 1)
        gemm_half(p + N_DEV - 1, 0)
        ra.wait()
        accumulate(0, 0, p + N_DEV - 1)
        ra = rs_rdma(0, 1, p + N_DEV - 1)
        ra.start()

        prep_chunk(p + 1)
        gemm_half(p + 1, 1)
        rb.wait()
        accumulate(1, 0, p + 1)
        rb = rs_rdma(1, 1, p + 1)
        rb.start()

        prep_chunk(p + 2)
        gemm_half(p + 2, 0)
        ra.wait()
        accumulate(0, 1, p + 2)
        ra = rs_rdma(0, 2, p + 2)
        ra.start()

        gemm_half(p + 2, 1)
        rb.wait()
        accumulate(1, 1, p + 2)
        rb = rs_rdma(1, 2, p + 2)
        rb.start()

        gemm_half(p + 1, 0)
        ra.wait()
        accumulate(0, 2, p + 1)
        for u in range(N_SUB):
            ag_start(0, 0, u, p + N_DEV + 1)

        gemm_half(p + N_DEV - 1, 1)
        rb.wait()
        accumulate(1, 2, p + N_DEV - 1)
        for u in range(N_SUB):
            ag_start(1, 0, u, p + N_DEV - 1)

        for t in range(1, N_RS):
            for u in range(N_SUB):
                ag_rdma(0, t - 1, u, p + N_DEV - (t - 1)).wait_recv()
                ag_start(0, t, u, p + N_DEV + 1 - t)
                ag_rdma(1, t - 1, u, p + (t - 1)).wait_recv()
                ag_start(1, t, u, p + N_DEV - 1 + t)
        for u in range(N_SUB):
            ag_rdma(0, N_RS - 1, u, p + N_DEV - (N_RS - 1)).wait_recv()
            ag_rdma(1, N_RS - 1, u, p + (N_RS - 1)).wait_recv()
        for r in ag_sends:
            r.wait_send()

        @functools.partial(pl.run_scoped, sem2=pltpu.SemaphoreType.REGULAR)
        def _(sem2):
            for nbr in (left, right):
                pl.semaphore_signal(sem2, inc=1, device_id=(nbr,),
                                    device_id_type=pl.DeviceIdType.MESH)
            pl.semaphore_wait(sem2, 2)

    return pl.pallas_call(
        body,
        out_shape=jax.ShapeDtypeStruct((N_TOK, D_FF), jnp.bfloat16),
        in_specs=[pl.BlockSpec(memory_space=pltpu.VMEM)] * 4,
        out_specs=pl.BlockSpec(memory_space=pltpu.VMEM),
        scratch_shapes=[
            pltpu.VMEM((N_TOK, E_LOCAL * D_MODEL), jnp.bfloat16),
            pltpu.VMEM((2, N_RS, CHUNK, HALF), jnp.bfloat16),
            pltpu.SemaphoreType.DMA((2, N_SEM)),
            pltpu.SemaphoreType.DMA((2, N_SEM)),
        ],
        compiler_params=pltpu.CompilerParams(collective_id=0),
    )(x, router_W, route_idx, expert_W)
